# Initial kernel scaffold; baseline (speedup 1.0000x reference)
import jax
import jax.numpy as jnp
from jax.experimental import pallas as pl


def kernel(edge_feat, edges, edge_classes, detector_labels):
    raise NotImplementedError("write your pallas kernel here")


# speedup vs baseline: 4.5233x; 4.5233x over previous
import functools

import jax
import jax.numpy as jnp
from jax import lax
from jax.experimental import pallas as pl
from jax.experimental.pallas import tpu as pltpu
from jax.experimental.pallas import tpu_sc as plsc

_NC = 2
_NS = 16
_NW = _NC * _NS
_L = 16


def _sc_mask_call(eflat, cflat, labels, E):
    N = labels.shape[0]
    CH = E // _NW
    CH2 = 2 * CH

    mesh = plsc.VectorSubcoreMesh(core_axis_name="c", subcore_axis_name="s")

    @functools.partial(
        pl.kernel,
        mesh=mesh,
        compiler_params=pltpu.CompilerParams(needs_layout_passes=False),
        out_type=(
            jax.ShapeDtypeStruct((E,), jnp.float32),
            jax.ShapeDtypeStruct((2 * E,), jnp.int32),
            jax.ShapeDtypeStruct((2 * E,), jnp.float32),
        ),
        scratch_types=[
            pltpu.VMEM((N,), jnp.int32),
            pltpu.VMEM((CH2,), jnp.int32),
            pltpu.VMEM((CH2,), jnp.float32),
            pltpu.VMEM((CH2,), jnp.float32),
            pltpu.VMEM((CH,), jnp.float32),
        ],
    )
    def sc_kern(eflat_hbm, cflat_hbm, labels_hbm, mask_hbm, eout_hbm, cout_hbm,
                labels_v, ev, cv, kv, mv):
        wid = lax.axis_index("s") * _NC + lax.axis_index("c")
        b1 = wid * CH
        b2 = wid * CH2
        pltpu.sync_copy(labels_hbm, labels_v)
        pltpu.sync_copy(eflat_hbm.at[pl.ds(b2, CH2)], ev)
        pltpu.sync_copy(cflat_hbm.at[pl.ds(b2, CH2)], cv)

        iota = lax.iota(jnp.int32, _L)
        ieven = iota & jnp.int32(-2)

        def body1(i, carry):
            off = pl.multiple_of(i * _L, _L)
            idx = ev[pl.ds(off, _L)]
            j0 = off + ieven
            a = plsc.load_gather(ev, [j0])
            b = plsc.load_gather(ev, [j0 + 1])
            l0 = plsc.load_gather(labels_v, [a])
            l1 = plsc.load_gather(labels_v, [b])
            keep = l0 & l1
            kv[pl.ds(off, _L)] = keep.astype(jnp.float32)
            ev[pl.ds(off, _L)] = idx * keep
            return carry

        lax.fori_loop(0, CH2 // _L, body1, 0)

        def body2(i, carry):
            off = pl.multiple_of(i * _L, _L)
            cv[pl.ds(off, _L)] = cv[pl.ds(off, _L)] * kv[pl.ds(off, _L)]
            return carry

        lax.fori_loop(0, CH2 // _L, body2, 0)

        def body3(i, carry):
            off = pl.multiple_of(i * _L, _L)
            m = plsc.load_gather(kv, [2 * off + 2 * iota])
            mv[pl.ds(off, _L)] = m
            return carry

        lax.fori_loop(0, CH // _L, body3, 0)

        pltpu.sync_copy(mv, mask_hbm.at[pl.ds(b1, CH)])
        pltpu.sync_copy(ev, eout_hbm.at[pl.ds(b2, CH2)])
        pltpu.sync_copy(cv, cout_hbm.at[pl.ds(b2, CH2)])

    return sc_kern(eflat, cflat, labels)


def _tc_body(m_ref, x_ref, o_ref):
    o_ref[...] = x_ref[...] * m_ref[...]


def _tc_mask_call(edge_feat, mask2d, E, D):
    B = 6400
    G = E // B
    return pl.pallas_call(
        _tc_body,
        grid=(G,),
        in_specs=[
            pl.BlockSpec((B, 1), lambda i: (i, 0)),
            pl.BlockSpec((B, D), lambda i: (i, 0)),
        ],
        out_specs=pl.BlockSpec((B, D), lambda i: (i, 0)),
        out_shape=jax.ShapeDtypeStruct((E, D), jnp.float32),
    )(mask2d, edge_feat)


def kernel(edge_feat, edges, edge_classes, detector_labels):
    E, D = edge_feat.shape
    edges_i = edges.astype(jnp.int32)
    eflat = edges_i.reshape(-1)
    cflat = edge_classes.reshape(-1)
    labels = detector_labels.astype(jnp.int32)

    mask, eout_flat, cout_flat = _sc_mask_call(eflat, cflat, labels, E)
    feat_out = _tc_mask_call(edge_feat, mask.reshape(E, 1), E, D)

    return (
        feat_out,
        eout_flat.reshape(E, 2).astype(edges.dtype),
        cout_flat.reshape(E, 2),
    )


# speedup vs baseline: 4.7876x; 1.0584x over previous
import functools

import jax
import jax.numpy as jnp
from jax import lax
from jax.experimental import pallas as pl
from jax.experimental.pallas import tpu as pltpu
from jax.experimental.pallas import tpu_sc as plsc

_NC = 2
_NS = 16
_NW = _NC * _NS
_L = 16


def _sc_call(eflat, cflat, feat, labels, E, D):
    N = labels.shape[0]
    CH = E // _NW
    CH2 = 2 * CH
    FB = 200
    NFB = CH // FB

    mesh = plsc.VectorSubcoreMesh(core_axis_name="c", subcore_axis_name="s")

    @functools.partial(
        pl.kernel,
        mesh=mesh,
        compiler_params=pltpu.CompilerParams(needs_layout_passes=False),
        out_type=(
            jax.ShapeDtypeStruct((E, D), jnp.float32),
            jax.ShapeDtypeStruct((2 * E,), jnp.int32),
            jax.ShapeDtypeStruct((2 * E,), jnp.float32),
        ),
        scratch_types=[
            pltpu.VMEM((N,), jnp.int32),
            pltpu.VMEM((CH2,), jnp.int32),
            pltpu.VMEM((CH2,), jnp.float32),
            pltpu.VMEM((CH,), jnp.int32),
            pltpu.VMEM((2, FB, D), jnp.float32),
        ],
    )
    def sc_kern(eflat_hbm, cflat_hbm, feat_hbm, labels_hbm,
                feat_out_hbm, eout_hbm, cout_hbm,
                labels_v, ev, cv, mv, fv):
        wid = lax.axis_index("s") * _NC + lax.axis_index("c")
        b1 = wid * CH
        b2 = wid * CH2
        pltpu.sync_copy(labels_hbm, labels_v)
        pltpu.sync_copy(eflat_hbm.at[pl.ds(b2, CH2)], ev)
        pltpu.sync_copy(cflat_hbm.at[pl.ds(b2, CH2)], cv)

        iota = lax.iota(jnp.int32, _L)
        ihalf = lax.shift_right_logical(iota, 1)

        def body_mask(i, carry):
            off = pl.multiple_of(i * _L, _L)
            pos = 2 * off + 2 * iota
            a = plsc.load_gather(ev, [pos])
            b = plsc.load_gather(ev, [pos + 1])
            l0 = plsc.load_gather(labels_v, [a])
            l1 = plsc.load_gather(labels_v, [b])
            mv[pl.ds(off, _L)] = l0 & l1
            return carry

        lax.fori_loop(0, CH // _L, body_mask, 0)

        def body_ec(i, carry):
            off = pl.multiple_of(i * _L, _L)
            kd = plsc.load_gather(mv, [(off >> 1) + ihalf])
            ev[pl.ds(off, _L)] = ev[pl.ds(off, _L)] * kd
            cv[pl.ds(off, _L)] = cv[pl.ds(off, _L)] * kd.astype(jnp.float32)
            return carry

        lax.fori_loop(0, CH2 // _L, body_ec, 0)

        pltpu.sync_copy(ev, eout_hbm.at[pl.ds(b2, CH2)])
        pltpu.sync_copy(cv, cout_hbm.at[pl.ds(b2, CH2)])

        def body_feat(blk, carry):
            row0 = pl.multiple_of(b1 + blk * FB, 8)
            buf = blk % 2
            pltpu.sync_copy(feat_hbm.at[pl.ds(row0, FB)], fv.at[buf])

            def body_row(e, c2):
                k = plsc.load_gather(mv, [jnp.full((_L,), blk * FB, jnp.int32) + e])
                kf = k.astype(jnp.float32)
                for j in range(D // _L):
                    fv[buf, e, pl.ds(j * _L, _L)] = (
                        fv[buf, e, pl.ds(j * _L, _L)] * kf
                    )
                return c2

            lax.fori_loop(0, FB, body_row, 0)
            pltpu.sync_copy(fv.at[buf], feat_out_hbm.at[pl.ds(row0, FB)])
            return carry

        lax.fori_loop(0, NFB, body_feat, 0)

    return sc_kern(eflat, cflat, feat, labels)


def kernel(edge_feat, edges, edge_classes, detector_labels):
    E, D = edge_feat.shape
    eflat = edges.astype(jnp.int32).reshape(-1)
    cflat = edge_classes.reshape(-1)
    labels = detector_labels.astype(jnp.int32)

    feat_out, eout_flat, cout_flat = _sc_call(eflat, cflat, edge_feat, labels, E, D)

    return (
        feat_out,
        eout_flat.reshape(E, 2).astype(edges.dtype),
        cout_flat.reshape(E, 2),
    )


# speedup vs baseline: 16.7163x; 3.4916x over previous
import functools

import jax
import jax.numpy as jnp
from jax import lax
from jax.experimental import pallas as pl
from jax.experimental.pallas import tpu as pltpu
from jax.experimental.pallas import tpu_sc as plsc

_NC = 2
_NS = 16
_NW = _NC * _NS
_L = 16


def _sc_call(e0, e1, c0, c1, feat, labels, E, D):
    N = labels.shape[0]
    CH = E // _NW
    FB = 200
    NFB = CH // FB

    mesh = plsc.VectorSubcoreMesh(core_axis_name="c", subcore_axis_name="s")

    @functools.partial(
        pl.kernel,
        mesh=mesh,
        compiler_params=pltpu.CompilerParams(needs_layout_passes=False),
        out_type=(
            jax.ShapeDtypeStruct((E, D), jnp.float32),
            jax.ShapeDtypeStruct((E,), jnp.int32),
            jax.ShapeDtypeStruct((E,), jnp.int32),
            jax.ShapeDtypeStruct((E,), jnp.float32),
            jax.ShapeDtypeStruct((E,), jnp.float32),
        ),
        scratch_types=[
            pltpu.VMEM((N,), jnp.int32),
            pltpu.VMEM((CH,), jnp.int32),
            pltpu.VMEM((CH,), jnp.int32),
            pltpu.VMEM((CH,), jnp.float32),
            pltpu.VMEM((CH,), jnp.float32),
            pltpu.VMEM((CH,), jnp.int32),
            pltpu.VMEM((2, FB, D), jnp.float32),
        ],
    )
    def sc_kern(e0_hbm, e1_hbm, c0_hbm, c1_hbm, feat_hbm, labels_hbm,
                feat_out_hbm, e0o_hbm, e1o_hbm, c0o_hbm, c1o_hbm,
                labels_v, e0v, e1v, c0v, c1v, mv, fv):
        wid = lax.axis_index("s") * _NC + lax.axis_index("c")
        b1 = wid * CH
        pltpu.sync_copy(labels_hbm, labels_v)
        pltpu.sync_copy(e0_hbm.at[pl.ds(b1, CH)], e0v)
        pltpu.sync_copy(e1_hbm.at[pl.ds(b1, CH)], e1v)
        pltpu.sync_copy(c0_hbm.at[pl.ds(b1, CH)], c0v)
        pltpu.sync_copy(c1_hbm.at[pl.ds(b1, CH)], c1v)

        def body_mask(i, carry):
            off = pl.multiple_of(i * _L, _L)
            l0 = plsc.load_gather(labels_v, [e0v[pl.ds(off, _L)]])
            l1 = plsc.load_gather(labels_v, [e1v[pl.ds(off, _L)]])
            k = l0 & l1
            kf = k.astype(jnp.float32)
            mv[pl.ds(off, _L)] = k
            e0v[pl.ds(off, _L)] = e0v[pl.ds(off, _L)] * k
            e1v[pl.ds(off, _L)] = e1v[pl.ds(off, _L)] * k
            c0v[pl.ds(off, _L)] = c0v[pl.ds(off, _L)] * kf
            c1v[pl.ds(off, _L)] = c1v[pl.ds(off, _L)] * kf
            return carry

        lax.fori_loop(0, CH // _L, body_mask, 0)

        pltpu.sync_copy(e0v, e0o_hbm.at[pl.ds(b1, CH)])
        pltpu.sync_copy(e1v, e1o_hbm.at[pl.ds(b1, CH)])
        pltpu.sync_copy(c0v, c0o_hbm.at[pl.ds(b1, CH)])
        pltpu.sync_copy(c1v, c1o_hbm.at[pl.ds(b1, CH)])

        def body_feat(blk, carry):
            row0 = pl.multiple_of(b1 + blk * FB, 8)
            buf = blk % 2
            pltpu.sync_copy(feat_hbm.at[pl.ds(row0, FB)], fv.at[buf])

            def body_row(e, c2):
                k = plsc.load_gather(mv, [jnp.full((_L,), blk * FB, jnp.int32) + e])
                kf = k.astype(jnp.float32)
                for j in range(D // _L):
                    fv[buf, e, pl.ds(j * _L, _L)] = (
                        fv[buf, e, pl.ds(j * _L, _L)] * kf
                    )
                return c2

            lax.fori_loop(0, FB, body_row, 0)
            pltpu.sync_copy(fv.at[buf], feat_out_hbm.at[pl.ds(row0, FB)])
            return carry

        lax.fori_loop(0, NFB, body_feat, 0)

    return sc_kern(e0, e1, c0, c1, feat, labels)


def kernel(edge_feat, edges, edge_classes, detector_labels):
    E, D = edge_feat.shape
    edges_i = edges.astype(jnp.int32)
    labels = detector_labels.astype(jnp.int32)

    feat_out, e0o, e1o, c0o, c1o = _sc_call(
        edges_i[:, 0], edges_i[:, 1],
        edge_classes[:, 0], edge_classes[:, 1],
        edge_feat, labels, E, D,
    )

    return (
        feat_out,
        jnp.stack([e0o, e1o], axis=1).astype(edges.dtype),
        jnp.stack([c0o, c1o], axis=1),
    )


# speedup vs baseline: 24.9835x; 1.4946x over previous
import functools

import jax
import jax.numpy as jnp
from jax import lax
from jax.experimental import pallas as pl
from jax.experimental.pallas import tpu as pltpu
from jax.experimental.pallas import tpu_sc as plsc

_NC = 2
_NS = 16
_NW = _NC * _NS
_L = 16
_NB = 5


def _sc_call(e0, e1, c0, c1, feat, labels, E, D):
    N = labels.shape[0]
    CH = E // _NW
    FB = 80
    NFB = CH // FB

    mesh = plsc.VectorSubcoreMesh(core_axis_name="c", subcore_axis_name="s")

    @functools.partial(
        pl.kernel,
        mesh=mesh,
        compiler_params=pltpu.CompilerParams(needs_layout_passes=False),
        out_type=(
            jax.ShapeDtypeStruct((E, D), jnp.float32),
            jax.ShapeDtypeStruct((E,), jnp.int32),
            jax.ShapeDtypeStruct((E,), jnp.int32),
            jax.ShapeDtypeStruct((E,), jnp.float32),
            jax.ShapeDtypeStruct((E,), jnp.float32),
        ),
        scratch_types=[
            pltpu.VMEM((N,), jnp.int32),
            pltpu.VMEM((CH,), jnp.int32),
            pltpu.VMEM((CH,), jnp.int32),
            pltpu.VMEM((CH,), jnp.float32),
            pltpu.VMEM((CH,), jnp.float32),
            pltpu.VMEM((CH,), jnp.float32),
            pltpu.VMEM((_NB, FB, D), jnp.float32),
            pltpu.SemaphoreType.DMA,
            [pltpu.SemaphoreType.DMA] * _NB,
            [pltpu.SemaphoreType.DMA] * _NB,
        ],
    )
    def sc_kern(e0_hbm, e1_hbm, c0_hbm, c1_hbm, feat_hbm, labels_hbm,
                feat_out_hbm, e0o_hbm, e1o_hbm, c0o_hbm, c1o_hbm,
                labels_v, e0v, e1v, c0v, c1v, mv, fv,
                sem_col, sems_in, sems_out):
        wid = lax.axis_index("s") * _NC + lax.axis_index("c")
        b1 = wid * CH

        h0 = pltpu.async_copy(labels_hbm, labels_v, sem_col)
        h1 = pltpu.async_copy(e0_hbm.at[pl.ds(b1, CH)], e0v, sem_col)
        h2 = pltpu.async_copy(e1_hbm.at[pl.ds(b1, CH)], e1v, sem_col)
        h3 = pltpu.async_copy(c0_hbm.at[pl.ds(b1, CH)], c0v, sem_col)
        h4 = pltpu.async_copy(c1_hbm.at[pl.ds(b1, CH)], c1v, sem_col)
        h0.wait()
        h1.wait()
        h2.wait()
        h3.wait()
        h4.wait()

        def body_mask(i, carry):
            off = pl.multiple_of(i * _L, _L)
            l0 = plsc.load_gather(labels_v, [e0v[pl.ds(off, _L)]])
            l1 = plsc.load_gather(labels_v, [e1v[pl.ds(off, _L)]])
            k = l0 & l1
            kf = k.astype(jnp.float32)
            mv[pl.ds(off, _L)] = kf
            e0v[pl.ds(off, _L)] = e0v[pl.ds(off, _L)] * k
            e1v[pl.ds(off, _L)] = e1v[pl.ds(off, _L)] * k
            c0v[pl.ds(off, _L)] = c0v[pl.ds(off, _L)] * kf
            c1v[pl.ds(off, _L)] = c1v[pl.ds(off, _L)] * kf
            return carry

        lax.fori_loop(0, CH // _L, body_mask, 0)

        hc0 = pltpu.async_copy(e0v, e0o_hbm.at[pl.ds(b1, CH)], sem_col)
        hc1 = pltpu.async_copy(e1v, e1o_hbm.at[pl.ds(b1, CH)], sem_col)
        hc2 = pltpu.async_copy(c0v, c0o_hbm.at[pl.ds(b1, CH)], sem_col)
        hc3 = pltpu.async_copy(c1v, c1o_hbm.at[pl.ds(b1, CH)], sem_col)

        def in_slice(blk):
            row0 = pl.multiple_of(b1 + blk * FB, 8)
            return feat_hbm.at[pl.ds(row0, FB)]

        def out_slice(blk):
            row0 = pl.multiple_of(b1 + blk * FB, 8)
            return feat_out_hbm.at[pl.ds(row0, FB)]

        for b in range(_NB):
            pltpu.async_copy(in_slice(b), fv.at[b], sems_in[b])

        def round_body(g, carry):
            blk0 = g * _NB
            for b in range(_NB):
                blk = blk0 + b
                pltpu.make_async_copy(in_slice(blk), fv.at[b], sems_in[b]).wait()

                basev = jnp.full((_L,), blk * FB, jnp.int32)

                def body_row(e, c2):
                    kf = plsc.load_gather(mv, [basev + e])
                    for j in range(D // _L):
                        fv[b, e, pl.ds(j * _L, _L)] = (
                            fv[b, e, pl.ds(j * _L, _L)] * kf
                        )
                    return c2

                lax.fori_loop(0, FB, body_row, 0)
                pltpu.async_copy(fv.at[b], out_slice(blk), sems_out[b])

                rblk = blk + 2
                rb = (b + 2) % _NB

                @pl.when(jnp.logical_and(rblk >= _NB, rblk < NFB))
                def _():
                    pltpu.make_async_copy(
                        fv.at[rb], out_slice(rblk - _NB), sems_out[rb]
                    ).wait()
                    pltpu.async_copy(in_slice(rblk), fv.at[rb], sems_in[rb])

            return carry

        lax.fori_loop(0, NFB // _NB, round_body, 0)

        for b in range(_NB):
            blk = NFB - _NB + b
            pltpu.make_async_copy(fv.at[b], out_slice(blk), sems_out[b]).wait()
        hc0.wait()
        hc1.wait()
        hc2.wait()
        hc3.wait()

    return sc_kern(e0, e1, c0, c1, feat, labels)


def kernel(edge_feat, edges, edge_classes, detector_labels):
    E, D = edge_feat.shape
    edges_i = edges.astype(jnp.int32)
    labels = detector_labels.astype(jnp.int32)

    feat_out, e0o, e1o, c0o, c1o = _sc_call(
        edges_i[:, 0], edges_i[:, 1],
        edge_classes[:, 0], edge_classes[:, 1],
        edge_feat, labels, E, D,
    )

    return (
        feat_out,
        jnp.stack([e0o, e1o], axis=1).astype(edges.dtype),
        jnp.stack([c0o, c1o], axis=1),
    )
